# baseline (speedup 1.0000x reference)
import functools
import jax
import jax.numpy as jnp
from jax import lax
from jax.experimental import pallas as pl
from jax.experimental.pallas import tpu as pltpu
from jax.experimental.pallas import tpu_sc as plsc

H = 128
LANES = 16
HV = H // LANES
NC = 2
NS = 16
NW = NC * NS
C = 128
EPS = 1e-12


def _rsqrt(v):
    bits = lax.bitcast_convert_type(v, jnp.int32)
    y = lax.bitcast_convert_type(
        jnp.int32(0x5F3759DF) - lax.shift_right_logical(bits, 1), jnp.float32)
    half = 0.5 * v
    for _ in range(3):
        y = y * (1.5 - half * y * y)
    return y


_GATHER_DNUMS = lax.GatherDimensionNumbers(
    offset_dims=(), collapsed_slice_dims=(0,), start_index_map=(0,))


def _lane_shuffle(x, idx):
    return lax.gather(x, idx[:, None], _GATHER_DNUMS, (1,),
                      mode=lax.GatherScatterMode.PROMISE_IN_BOUNDS)


def _hsum(x):
    lanes = lax.iota(jnp.int32, LANES)
    for sh in (8, 4, 2, 1):
        x = x + _lane_shuffle(x, lanes ^ sh)
    return x


def _sc_fused(tgt_idx, msa_idx, table, pos_tab, gamma, beta, B, L, n_msa):
    total = B * L
    P = total // NW
    n_chunks = P // C
    n_steps = n_chunks * n_msa
    inv_n = 1.0 / n_msa
    inv_h = 1.0 / H

    mesh = plsc.VectorSubcoreMesh(core_axis_name="c", subcore_axis_name="s")

    @functools.partial(
        pl.kernel,
        out_type=(
            jax.ShapeDtypeStruct((total, H), jnp.float32),
            jax.ShapeDtypeStruct((total, H), jnp.float32),
        ),
        mesh=mesh,
        scratch_types=[
            pltpu.VMEM((P,), jnp.int32),
            pltpu.VMEM((n_msa * P,), jnp.int32),
            pltpu.VMEM((P, H), jnp.float32),
            pltpu.VMEM((C, H), jnp.float32),
            pltpu.VMEM((C, H), jnp.float32),
            pltpu.VMEM((P, H), jnp.float32),
            pltpu.VMEM((C, H), jnp.float32),
            pltpu.VMEM((H,), jnp.float32),
            pltpu.VMEM((H,), jnp.float32),
            pltpu.SemaphoreType.DMA,
            pltpu.SemaphoreType.DMA,
            pltpu.SemaphoreType.DMA,
            pltpu.SemaphoreType.DMA,
            pltpu.SemaphoreType.DMA,
            pltpu.SemaphoreType.DMA,
            pltpu.SemaphoreType.DMA,
        ],
    )
    def k(tgt_idx_hbm, msa_idx_hbm, table_hbm, pos_hbm, gamma_hbm, beta_hbm,
          emb_out, m_out,
          tidx, midx, trows, buf0, buf1, acc, pbuf, gvec, bvec,
          sem_i, sem_t, sem_a, sem_b0, sem_b1, sem_p, sem_o):
        wid = lax.axis_index("s") * NC + lax.axis_index("c")
        base = wid * P
        b = base // L
        l0 = base - b * L
        src0 = b * (n_msa * L) + l0

        icps = [
            pltpu.make_async_copy(tgt_idx_hbm.at[pl.ds(base, P)], tidx, sem_i),
            pltpu.make_async_copy(gamma_hbm, gvec, sem_i),
            pltpu.make_async_copy(beta_hbm, bvec, sem_i),
        ]
        for j in range(n_msa):
            icps.append(pltpu.make_async_copy(
                msa_idx_hbm.at[pl.ds(src0 + j * L, P)],
                midx.at[pl.ds(j * P, P)], sem_i))
        for cp in icps:
            cp.start()

        pcp = pltpu.make_async_copy(pos_hbm.at[pl.ds(l0, C)], pbuf, sem_p)
        pcp.start()

        for cp in icps:
            cp.wait()

        tcps = []
        for ci in range(n_chunks):
            cp = pltpu.make_async_copy(
                table_hbm.at[tidx.at[pl.ds(ci * C, C)]],
                trows.at[pl.ds(ci * C, C)], sem_t)
            cp.start()
            tcps.append(cp)

        bufs = (buf0, buf1)
        bsems = (sem_b0, sem_b1)
        ocps = []

        def fire(step):
            ci, j = divmod(step, n_msa)
            isl = midx.at[pl.ds(j * P + ci * C, C)]
            if j == 0:
                cp = pltpu.make_async_copy(
                    table_hbm.at[isl], acc.at[pl.ds(ci * C, C)], sem_a)
            else:
                cp = pltpu.make_async_copy(
                    table_hbm.at[isl], bufs[step % 2], bsems[step % 2])
            cp.start()
            return cp

        def finalize(ci):
            tcps[ci].wait()
            pcp.wait()
            a0 = ci * C

            def fin_row(p, _):
                r = a0 + p
                xs = []
                s16 = jnp.zeros((LANES,), jnp.float32)
                q16 = jnp.zeros((LANES,), jnp.float32)
                for h in range(HV):
                    hs = pl.ds(h * LANES, LANES)
                    m = acc[r, hs] * inv_n
                    acc[r, hs] = m
                    x = trows[r, hs] + m + pbuf[p, hs]
                    xs.append(x)
                    s16 = s16 + x
                    q16 = q16 + x * x
                mean = _hsum(s16) * inv_h
                var = _hsum(q16) * inv_h - mean * mean
                inv = _rsqrt(var + EPS)
                for h in range(HV):
                    hs = pl.ds(h * LANES, LANES)
                    trows[r, hs] = (xs[h] - mean) * inv * gvec[hs] + bvec[hs]
                return 0

            lax.fori_loop(0, C, fin_row, 0)
            for ref, out in ((trows, emb_out), (acc, m_out)):
                cp = pltpu.make_async_copy(
                    ref.at[pl.ds(a0, C)], out.at[pl.ds(base + a0, C)], sem_o)
                cp.start()
                ocps.append(cp)

        cps = {0: fire(0), 1: fire(1)}
        for step in range(n_steps):
            cps.pop(step).wait()
            ci, j = divmod(step, n_msa)
            if j > 0:
                src = bufs[step % 2]
                a0 = ci * C

                def add_row(p, _):
                    for h in range(HV):
                        hs = pl.ds(h * LANES, LANES)
                        plsc.addupdate(acc.at[a0 + p, hs], src[p, hs])
                    return 0

                lax.fori_loop(0, C, add_row, 0)
            if step + 2 < n_steps:
                cps[step + 2] = fire(step + 2)
            if j == n_msa - 1:
                finalize(ci)
                if ci + 1 < n_chunks:
                    pcp = pltpu.make_async_copy(
                        pos_hbm.at[pl.ds(l0 + (ci + 1) * C, C)], pbuf, sem_p)
                    pcp.start()

        for cp in ocps:
            cp.wait()

    return k(tgt_idx, msa_idx, table, pos_tab, gamma, beta)


def kernel(target_ids, input_ids, word_embeddings, position_embeddings, gamma, beta):
    B, L = target_ids.shape
    n_msa = input_ids.shape[1]
    tgt_idx = target_ids.astype(jnp.int32).reshape(-1)
    msa_idx = input_ids.astype(jnp.int32).reshape(-1)
    emb, msa_mean = _sc_fused(tgt_idx, msa_idx, word_embeddings,
                              position_embeddings, gamma, beta, B, L, n_msa)
    return emb.reshape(B, L, H), msa_mean.reshape(B, L, H)


# speedup vs baseline: 1.2865x; 1.2865x over previous
import functools
import jax
import jax.numpy as jnp
from jax import lax
from jax.experimental import pallas as pl
from jax.experimental.pallas import tpu as pltpu
from jax.experimental.pallas import tpu_sc as plsc

H = 128
LANES = 16
HV = H // LANES
NC = 2
NS = 16
NW = NC * NS
C = 128
NBUF = 3


def _sc_gather_pool(tgt_idx, msa_idx, table, B, L, n_msa):
    total = B * L
    P = total // NW
    n_chunks = P // C
    n_steps = n_chunks * n_msa
    inv_n = 1.0 / n_msa

    mesh = plsc.VectorSubcoreMesh(core_axis_name="c", subcore_axis_name="s")

    @functools.partial(
        pl.kernel,
        out_type=(
            jax.ShapeDtypeStruct((total, H), jnp.float32),
            jax.ShapeDtypeStruct((total, H), jnp.float32),
        ),
        mesh=mesh,
        scratch_types=[
            pltpu.VMEM((P,), jnp.int32),
            pltpu.VMEM((n_msa * P,), jnp.int32),
            pltpu.VMEM((P, H), jnp.float32),
            pltpu.VMEM((NBUF, C, H), jnp.float32),
            pltpu.VMEM((P, H), jnp.float32),
            pltpu.SemaphoreType.DMA,
            pltpu.SemaphoreType.DMA,
            pltpu.SemaphoreType.DMA,
            pltpu.SemaphoreType.DMA,
            pltpu.SemaphoreType.DMA,
            pltpu.SemaphoreType.DMA,
            pltpu.SemaphoreType.DMA,
        ],
    )
    def k(tgt_idx_hbm, msa_idx_hbm, table_hbm, s_out, m_out,
          tidx, midx, trows, ring, acc,
          sem_i, sem_t, sem_a, sem_b0, sem_b1, sem_b2, sem_o):
        wid = lax.axis_index("s") * NC + lax.axis_index("c")
        base = wid * P
        b = base // L
        l0 = base - b * L
        src0 = b * (n_msa * L) + l0
        bsems = (sem_b0, sem_b1, sem_b2)

        icps = [pltpu.make_async_copy(
            tgt_idx_hbm.at[pl.ds(base, P)], tidx, sem_i)]
        for j in range(n_msa):
            icps.append(pltpu.make_async_copy(
                msa_idx_hbm.at[pl.ds(src0 + j * L, P)],
                midx.at[pl.ds(j * P, P)], sem_i))
        for cp in icps:
            cp.start()
        for cp in icps:
            cp.wait()

        tcps = []
        for ci in range(n_chunks):
            cp = pltpu.make_async_copy(
                table_hbm.at[tidx.at[pl.ds(ci * C, C)]],
                trows.at[pl.ds(ci * C, C)], sem_t)
            cp.start()
            tcps.append(cp)

        ocps = []

        def fire(step):
            ci, j = divmod(step, n_msa)
            isl = midx.at[pl.ds(j * P + ci * C, C)]
            if j == 0:
                cp = pltpu.make_async_copy(
                    table_hbm.at[isl], acc.at[pl.ds(ci * C, C)], sem_a)
            else:
                sl = step % NBUF
                cp = pltpu.make_async_copy(
                    table_hbm.at[isl], ring.at[sl], bsems[sl])
            cp.start()
            return cp

        def finalize(ci):
            tcps[ci].wait()
            a0 = ci * C

            def fin_row(p, _):
                r = a0 + p
                for h in range(HV):
                    hs = pl.ds(h * LANES, LANES)
                    m = acc[r, hs] * inv_n
                    acc[r, hs] = m
                    plsc.addupdate(trows.at[r, hs], m)
                return 0

            lax.fori_loop(0, C, fin_row, 0)
            for ref, out in ((trows, s_out), (acc, m_out)):
                cp = pltpu.make_async_copy(
                    ref.at[pl.ds(a0, C)], out.at[pl.ds(base + a0, C)], sem_o)
                cp.start()
                ocps.append(cp)

        cps = {s: fire(s) for s in range(min(NBUF, n_steps))}
        for step in range(n_steps):
            cps.pop(step).wait()
            ci, j = divmod(step, n_msa)
            if j > 0:
                src = ring.at[step % NBUF]
                a0 = ci * C

                def add_rows(i, _):
                    p = i * 2
                    for dp in range(2):
                        for h in range(HV):
                            hs = pl.ds(h * LANES, LANES)
                            plsc.addupdate(acc.at[a0 + p + dp, hs],
                                           src[p + dp, hs])
                    return 0

                lax.fori_loop(0, C // 2, add_rows, 0)
            if step + NBUF < n_steps:
                cps[step + NBUF] = fire(step + NBUF)
            if j == n_msa - 1:
                finalize(ci)

        for cp in ocps:
            cp.wait()

    return k(tgt_idx, msa_idx, table)


def _ln_body(s_ref, pos_ref, gamma_ref, beta_ref, out_ref):
    x = s_ref[0] + pos_ref[...]
    mean = jnp.mean(x, axis=-1, keepdims=True)
    cx = x - mean
    var = jnp.mean(cx * cx, axis=-1, keepdims=True)
    inv = lax.rsqrt(var + 1e-12)
    out_ref[0] = cx * inv * gamma_ref[0] + beta_ref[0]


def kernel(target_ids, input_ids, word_embeddings, position_embeddings, gamma, beta):
    B, L = target_ids.shape
    n_msa = input_ids.shape[1]
    tgt_idx = target_ids.astype(jnp.int32).reshape(-1)
    msa_idx = input_ids.astype(jnp.int32).reshape(-1)
    s, msa_mean = _sc_gather_pool(tgt_idx, msa_idx, word_embeddings, B, L, n_msa)
    pos = position_embeddings[:L]
    emb = pl.pallas_call(
        _ln_body,
        grid=(B,),
        in_specs=[
            pl.BlockSpec((1, L, H), lambda b: (b, 0, 0)),
            pl.BlockSpec((L, H), lambda b: (0, 0)),
            pl.BlockSpec((1, H), lambda b: (0, 0)),
            pl.BlockSpec((1, H), lambda b: (0, 0)),
        ],
        out_specs=pl.BlockSpec((1, L, H), lambda b: (b, 0, 0)),
        out_shape=jax.ShapeDtypeStruct((B, L, H), jnp.float32),
    )(s.reshape(B, L, H), pos, gamma.reshape(1, H), beta.reshape(1, H))
    return emb, msa_mean.reshape(B, L, H)
